# baseline (speedup 1.0000x reference)
import functools

import jax
import jax.numpy as jnp
from jax.experimental import pallas as pl
from jax.experimental.pallas import tpu as pltpu

_THR = 3.33
_BIG = 1e9


def _loss_body(ps_ref, pd_ref, fp_ref, ms_ref, md_ref, out_ref, u_ref, v_ref,
               *, n, ti):
    b = pl.program_id(0)

    pw = ps_ref[0] + fp_ref[0]
    pdm = pd_ref[0]
    sqw = jnp.sum(pw * pw, axis=0, keepdims=True)
    sqd = jnp.sum(pdm * pdm, axis=0, keepdims=True)
    ones = jnp.ones_like(sqw)
    zeros = jnp.zeros_like(sqw)
    pen_s = _BIG * (1.0 - ms_ref[0])
    pen_d = _BIG * (1.0 - md_ref[0])

    pwr = (-2.0 * pw).astype(jnp.bfloat16).astype(jnp.float32)
    pdr = pdm.astype(jnp.bfloat16).astype(jnp.float32)

    u_ref[...] = jnp.concatenate(
        [pwr, sqw, ones, pen_s, ones, zeros], axis=0)
    v_ref[...] = jnp.concatenate(
        [pdr, ones, sqd, ones, pen_d, zeros], axis=0)

    ni = n // ti

    def body(i, carry):
        sum_w, cnt_w, colmin = carry
        ut = u_ref[:, pl.ds(i * ti, ti)]
        dp = jax.lax.dot_general(
            ut, v_ref[...],
            dimension_numbers=(((0,), (0,)), ((), ())),
            preferred_element_type=jnp.float32,
            precision=jax.lax.Precision.HIGHEST)
        rmin = jnp.min(dp, axis=1, keepdims=True)
        sel = rmin < _THR
        sum_w = sum_w + jnp.sum(jnp.where(sel, rmin, 0.0))
        cnt_w = cnt_w + jnp.sum(sel.astype(jnp.float32))
        colmin = jnp.minimum(colmin, jnp.min(dp, axis=0, keepdims=True))
        return sum_w, cnt_w, colmin

    init = (jnp.float32(0.0), jnp.float32(0.0),
            jnp.full((1, n), jnp.inf, dtype=jnp.float32))
    sum_w, cnt_w, colmin = jax.lax.fori_loop(0, ni, body, init)

    sel_c = colmin < _THR
    sum_c = jnp.sum(jnp.where(sel_c, colmin, 0.0))
    cnt_c = jnp.sum(sel_c.astype(jnp.float32))
    loss_b = sum_w / cnt_w + sum_c / cnt_c

    @pl.when(b == 0)
    def _():
        out_ref[0, 0] = loss_b

    @pl.when(b != 0)
    def _():
        out_ref[0, 0] = out_ref[0, 0] + loss_b


def kernel(points_src, points_dst, flows_pred, flows_gt, masks_src, masks_dst):
    del flows_gt
    b, n, _ = points_src.shape
    ti = 256 if n % 256 == 0 else n

    ps = jnp.swapaxes(points_src, 1, 2)
    pd = jnp.swapaxes(points_dst, 1, 2)
    fp = jnp.swapaxes(flows_pred, 1, 2)
    ms = (masks_src > 0).astype(jnp.float32).reshape(b, 1, n)
    md = (masks_dst > 0).astype(jnp.float32).reshape(b, 1, n)

    point_spec = pl.BlockSpec((1, 3, n), lambda i: (i, 0, 0))
    mask_spec = pl.BlockSpec((1, 1, n), lambda i: (i, 0, 0))
    out = pl.pallas_call(
        functools.partial(_loss_body, n=n, ti=ti),
        grid=(b,),
        in_specs=[point_spec, point_spec, point_spec, mask_spec, mask_spec],
        out_specs=pl.BlockSpec((1, 1), lambda i: (0, 0),
                               memory_space=pltpu.SMEM),
        out_shape=jax.ShapeDtypeStruct((1, 1), jnp.float32),
        scratch_shapes=[pltpu.VMEM((8, n), jnp.float32),
                        pltpu.VMEM((8, n), jnp.float32)],
    )(ps, pd, fp, ms, md)
    return out[0, 0]


# speedup vs baseline: 4.3676x; 4.3676x over previous
import functools

import jax
import jax.numpy as jnp
from jax.experimental import pallas as pl
from jax.experimental.pallas import tpu as pltpu

_THR = 3.33
_BIG = 1e9


def _loss_body(ps_ref, pd_ref, fp_ref, ms_ref, md_ref, out_ref, u_ref, v_ref,
               *, n, ti):
    b = pl.program_id(0)

    pw = ps_ref[0] + fp_ref[0]
    pdm = pd_ref[0]
    sqw = jnp.sum(pw * pw, axis=0, keepdims=True)
    sqd = jnp.sum(pdm * pdm, axis=0, keepdims=True)
    ones = jnp.ones_like(sqw)
    zeros = jnp.zeros_like(sqw)
    pen_s = _BIG * (1.0 - ms_ref[0])
    pen_d = _BIG * (1.0 - md_ref[0])

    def _split3(x):
        hi = x.astype(jnp.bfloat16).astype(jnp.float32)
        r1 = x - hi
        mid = r1.astype(jnp.bfloat16).astype(jnp.float32)
        lo = r1 - mid
        return hi, mid, lo

    sqw_h, sqw_m, sqw_l = _split3(sqw)
    sqd_h, sqd_m, sqd_l = _split3(sqd)

    u_ref[...] = jnp.concatenate(
        [-2.0 * pw, sqw_h, sqw_m, sqw_l, ones, ones, ones, pen_s, ones,
         zeros, zeros, zeros, zeros, zeros], axis=0).astype(jnp.bfloat16)
    v_ref[...] = jnp.concatenate(
        [pdm, ones, ones, ones, sqd_h, sqd_m, sqd_l, ones, pen_d,
         zeros, zeros, zeros, zeros, zeros], axis=0).astype(jnp.bfloat16)

    ni = n // ti

    def body(i, carry):
        sum_w, cnt_w, colmin = carry
        ut = u_ref[:, pl.ds(i * ti, ti)]
        dp = jax.lax.dot_general(
            ut, v_ref[...],
            dimension_numbers=(((0,), (0,)), ((), ())),
            preferred_element_type=jnp.float32,
            precision=jax.lax.Precision.DEFAULT)
        rmin = jnp.min(dp, axis=1, keepdims=True)
        sel = rmin < _THR
        sum_w = sum_w + jnp.sum(jnp.where(sel, rmin, 0.0))
        cnt_w = cnt_w + jnp.sum(sel.astype(jnp.float32))
        colmin = jnp.minimum(colmin, jnp.min(dp, axis=0, keepdims=True))
        return sum_w, cnt_w, colmin

    init = (jnp.float32(0.0), jnp.float32(0.0),
            jnp.full((1, n), jnp.inf, dtype=jnp.float32))
    sum_w, cnt_w, colmin = jax.lax.fori_loop(0, ni, body, init)

    sel_c = colmin < _THR
    sum_c = jnp.sum(jnp.where(sel_c, colmin, 0.0))
    cnt_c = jnp.sum(sel_c.astype(jnp.float32))
    loss_b = sum_w / cnt_w + sum_c / cnt_c

    @pl.when(b == 0)
    def _():
        out_ref[0, 0] = loss_b

    @pl.when(b != 0)
    def _():
        out_ref[0, 0] = out_ref[0, 0] + loss_b


def kernel(points_src, points_dst, flows_pred, flows_gt, masks_src, masks_dst):
    del flows_gt
    b, n, _ = points_src.shape
    ti = 256 if n % 256 == 0 else n

    ps = jnp.swapaxes(points_src, 1, 2)
    pd = jnp.swapaxes(points_dst, 1, 2)
    fp = jnp.swapaxes(flows_pred, 1, 2)
    ms = (masks_src > 0).astype(jnp.float32).reshape(b, 1, n)
    md = (masks_dst > 0).astype(jnp.float32).reshape(b, 1, n)

    point_spec = pl.BlockSpec((1, 3, n), lambda i: (i, 0, 0))
    mask_spec = pl.BlockSpec((1, 1, n), lambda i: (i, 0, 0))
    out = pl.pallas_call(
        functools.partial(_loss_body, n=n, ti=ti),
        grid=(b,),
        in_specs=[point_spec, point_spec, point_spec, mask_spec, mask_spec],
        out_specs=pl.BlockSpec((1, 1), lambda i: (0, 0),
                               memory_space=pltpu.SMEM),
        out_shape=jax.ShapeDtypeStruct((1, 1), jnp.float32),
        scratch_shapes=[pltpu.VMEM((16, n), jnp.bfloat16),
                        pltpu.VMEM((16, n), jnp.bfloat16)],
    )(ps, pd, fp, ms, md)
    return out[0, 0]


# speedup vs baseline: 4.6359x; 1.0614x over previous
import functools

import jax
import jax.numpy as jnp
from jax import lax
from jax.experimental import pallas as pl
from jax.experimental.pallas import tpu as pltpu
from jax.experimental.pallas import tpu_sc as plsc

_THR = 3.33
_BIG = 1e9
_LANES = 16


def _compact_body(p_hbm, f_hbm, m_hbm, outc_hbm, cnt_hbm,
                  pb0, pb1, pb2, fb0, fb1, fb2, mb, ob0, ob1, ob2, cb, sb,
                  *, n):
    pbufs = (pb0, pb1, pb2)
    fbufs = (fb0, fb1, fb2)
    obufs = (ob0, ob1, ob2)
    lanes = lax.iota(jnp.int32, _LANES)

    def _cumsum16(s):
        for k in (1, 2, 4, 8):
            sb[...] = s
            g = plsc.load_gather(sb, [jnp.maximum(lanes - k, 0)])
            s = s + jnp.where(lanes >= k, g, 0)
        return s
    w = lax.axis_index("s") * 2 + lax.axis_index("c")

    @pl.when(w < 4)
    def _():
        for k in range(3):
            pltpu.sync_copy(p_hbm.at[pl.ds((w * 3 + k) * n, n)], pbufs[k])
            pltpu.sync_copy(f_hbm.at[pl.ds((w * 3 + k) * n, n)], fbufs[k])
        pltpu.sync_copy(m_hbm.at[pl.ds(w * n, n)], mb)

        def body(i, base):
            mv = mb[pl.ds(i * _LANES, _LANES)]
            valid = mv > 0
            pos = base + _cumsum16(mv) - 1
            for k in range(3):
                pv = (pbufs[k][pl.ds(i * _LANES, _LANES)]
                      + fbufs[k][pl.ds(i * _LANES, _LANES)])
                plsc.store_scatter(obufs[k], [pos], pv, mask=valid)
            return base + plsc.all_reduce_population_count(valid)

        cnt = lax.fori_loop(0, n // _LANES, body,
                            jnp.zeros((_LANES,), jnp.int32))
        cb[...] = cnt
        for k in range(3):
            pltpu.sync_copy(obufs[k], outc_hbm.at[pl.ds((w * 3 + k) * n, n)])
        pltpu.sync_copy(cb, cnt_hbm.at[pl.ds(w * _LANES, _LANES)])


def _compact(p, f, m, n):
    mesh = plsc.VectorSubcoreMesh(core_axis_name="c", subcore_axis_name="s")
    fvec = pltpu.VMEM((n,), jnp.float32)
    return pl.kernel(
        functools.partial(_compact_body, n=n),
        out_type=(jax.ShapeDtypeStruct((4 * 3 * n,), jnp.float32),
                  jax.ShapeDtypeStruct((4 * _LANES,), jnp.int32)),
        mesh=mesh,
        compiler_params=pltpu.CompilerParams(needs_layout_passes=False),
        scratch_types=[fvec, fvec, fvec, fvec, fvec, fvec,
                       pltpu.VMEM((n,), jnp.int32),
                       fvec, fvec, fvec,
                       pltpu.VMEM((_LANES,), jnp.int32),
                       pltpu.VMEM((_LANES,), jnp.int32)],
    )(p, f, m)


def _loss_body(cnt_ref, cw_ref, cd_ref, out_ref, u_ref, v_ref, colmin_ref,
               *, n, ti, tj):
    b = pl.program_id(0)
    c1 = cnt_ref[0, b]
    c2 = cnt_ref[0, 2 + b]

    iota = lax.broadcasted_iota(jnp.int32, (1, n), 1)
    mrow = iota < c1
    mcol = iota < c2
    pw = jnp.where(mrow, cw_ref[0], 0.0)
    pdm = jnp.where(mcol, cd_ref[0], 0.0)
    sqw = jnp.sum(pw * pw, axis=0, keepdims=True)
    sqd = jnp.sum(pdm * pdm, axis=0, keepdims=True)
    ones = jnp.ones_like(sqw)
    zeros = jnp.zeros_like(sqw)
    pen_s = jnp.where(mrow, 0.0, _BIG)
    pen_d = jnp.where(mcol, 0.0, _BIG)

    def _split3(x):
        hi = x.astype(jnp.bfloat16).astype(jnp.float32)
        r1 = x - hi
        mid = r1.astype(jnp.bfloat16).astype(jnp.float32)
        lo = r1 - mid
        return hi, mid, lo

    sqw_h, sqw_m, sqw_l = _split3(sqw)
    sqd_h, sqd_m, sqd_l = _split3(sqd)

    u_ref[...] = jnp.concatenate(
        [-2.0 * pw, sqw_h, sqw_m, sqw_l, ones, ones, ones, pen_s, ones,
         zeros, zeros, zeros, zeros, zeros], axis=0).astype(jnp.bfloat16)
    v_ref[...] = jnp.concatenate(
        [pdm, ones, ones, ones, sqd_h, sqd_m, sqd_l, ones, pen_d,
         zeros, zeros, zeros, zeros, zeros], axis=0).astype(jnp.bfloat16)

    colmin_ref[...] = jnp.full((1, n), jnp.inf, dtype=jnp.float32)

    ni = (c1 + ti - 1) // ti
    nj = (c2 + tj - 1) // tj

    def iloop(i, carry):
        sum_w, cnt_w = carry
        ut = u_ref[:, pl.ds(i * ti, ti)]

        def jloop(j, rm):
            vt = v_ref[:, pl.ds(j * tj, tj)]
            dp = lax.dot_general(
                ut, vt,
                dimension_numbers=(((0,), (0,)), ((), ())),
                preferred_element_type=jnp.float32,
                precision=lax.Precision.DEFAULT)
            cm = colmin_ref[:, pl.ds(j * tj, tj)]
            colmin_ref[:, pl.ds(j * tj, tj)] = jnp.minimum(
                cm, jnp.min(dp, axis=0, keepdims=True))
            return jnp.minimum(rm, jnp.min(dp, axis=1, keepdims=True))

        rmin = lax.fori_loop(0, nj, jloop,
                             jnp.full((ti, 1), jnp.inf, dtype=jnp.float32))
        sel = rmin < _THR
        sum_w = sum_w + jnp.sum(jnp.where(sel, rmin, 0.0))
        cnt_w = cnt_w + jnp.sum(sel.astype(jnp.float32))
        return sum_w, cnt_w

    sum_w, cnt_w = lax.fori_loop(0, ni, iloop,
                                 (jnp.float32(0.0), jnp.float32(0.0)))

    colmin = colmin_ref[...]
    sel_c = colmin < _THR
    sum_c = jnp.sum(jnp.where(sel_c, colmin, 0.0))
    cnt_c = jnp.sum(sel_c.astype(jnp.float32))
    loss_b = sum_w / cnt_w + sum_c / cnt_c

    @pl.when(b == 0)
    def _():
        out_ref[0, 0] = loss_b

    @pl.when(b != 0)
    def _():
        out_ref[0, 0] = out_ref[0, 0] + loss_b


def kernel(points_src, points_dst, flows_pred, flows_gt, masks_src, masks_dst):
    del flows_gt
    bsz, n, _ = points_src.shape
    ti, tj = 256, 2048

    ps = jnp.swapaxes(points_src, 1, 2)
    pd = jnp.swapaxes(points_dst, 1, 2)
    fp = jnp.swapaxes(flows_pred, 1, 2)

    p_all = jnp.concatenate([ps, pd], axis=0)
    f_all = jnp.concatenate([fp, jnp.zeros_like(pd)], axis=0)
    m_all = jnp.concatenate([masks_src.astype(jnp.int32),
                             masks_dst.astype(jnp.int32)], axis=0)

    comp_flat, cnt16 = _compact(p_all.reshape(-1), f_all.reshape(-1),
                                m_all.reshape(-1), n)
    comp = comp_flat.reshape(4, 3, n)
    cnts = cnt16.reshape(4, _LANES)[:, 0].reshape(1, 4)

    point_spec_w = pl.BlockSpec((1, 3, n), lambda i: (i, 0, 0))
    point_spec_d = pl.BlockSpec((1, 3, n), lambda i: (2 + i, 0, 0))
    out = pl.pallas_call(
        functools.partial(_loss_body, n=n, ti=ti, tj=tj),
        grid=(bsz,),
        in_specs=[
            pl.BlockSpec((1, 4), lambda i: (0, 0), memory_space=pltpu.SMEM),
            point_spec_w,
            point_spec_d,
        ],
        out_specs=pl.BlockSpec((1, 1), lambda i: (0, 0),
                               memory_space=pltpu.SMEM),
        out_shape=jax.ShapeDtypeStruct((1, 1), jnp.float32),
        scratch_shapes=[pltpu.VMEM((16, n), jnp.bfloat16),
                        pltpu.VMEM((16, n), jnp.bfloat16),
                        pltpu.VMEM((1, n), jnp.float32)],
    )(cnts, comp, comp)
    return out[0, 0]


# speedup vs baseline: 5.0565x; 1.0907x over previous
import functools

import jax
import jax.numpy as jnp
from jax import lax
from jax.experimental import pallas as pl
from jax.experimental.pallas import tpu as pltpu
from jax.experimental.pallas import tpu_sc as plsc

_THR = 3.33
_BIG = 1e9
_LANES = 16
_NSETS = 4
_NHALF = 2
_NW = _NSETS * _NHALF


def _compact_body(p_hbm, f_hbm, m_hbm, outc_hbm, cnt_hbm,
                  pb0, pb1, pb2, fb0, fb1, fb2, mb, ob0, ob1, ob2, cb, sb,
                  *, n):
    span = n // _NHALF
    pbufs = (pb0, pb1, pb2)
    fbufs = (fb0, fb1, fb2)
    obufs = (ob0, ob1, ob2)
    lanes = lax.iota(jnp.int32, _LANES)

    def _cumsum16(s):
        for k in (1, 2, 4, 8):
            sb[...] = s
            g = plsc.load_gather(sb, [jnp.maximum(lanes - k, 0)])
            s = s + jnp.where(lanes >= k, g, 0)
        return s

    w = lax.axis_index("s") * 2 + lax.axis_index("c")

    @pl.when(w < _NW)
    def _():
        for k in range(3):
            pltpu.sync_copy(
                p_hbm.at[pl.ds((w // _NHALF) * 3 * n + k * n
                               + (w % _NHALF) * span, span)], pbufs[k])
            pltpu.sync_copy(
                f_hbm.at[pl.ds((w // _NHALF) * 3 * n + k * n
                               + (w % _NHALF) * span, span)], fbufs[k])
        pltpu.sync_copy(
            m_hbm.at[pl.ds((w // _NHALF) * n + (w % _NHALF) * span, span)],
            mb)

        def body(i, base):
            mv = mb[pl.ds(i * _LANES, _LANES)]
            valid = mv > 0
            pos = base + _cumsum16(mv) - 1
            for k in range(3):
                pv = (pbufs[k][pl.ds(i * _LANES, _LANES)]
                      + fbufs[k][pl.ds(i * _LANES, _LANES)])
                plsc.store_scatter(obufs[k], [pos], pv, mask=valid)
            return base + plsc.all_reduce_population_count(valid)

        cnt = lax.fori_loop(0, span // _LANES, body,
                            jnp.zeros((_LANES,), jnp.int32))
        cb[...] = cnt
        for k in range(3):
            pltpu.sync_copy(
                obufs[k],
                outc_hbm.at[pl.ds((w // _NHALF) * 3 * n + k * n
                                  + (w % _NHALF) * span, span)])
        pltpu.sync_copy(cb, cnt_hbm.at[pl.ds(w * _LANES, _LANES)])


def _compact(p, f, m, n):
    span = n // _NHALF
    mesh = plsc.VectorSubcoreMesh(core_axis_name="c", subcore_axis_name="s")
    fvec = pltpu.VMEM((span,), jnp.float32)
    return pl.kernel(
        functools.partial(_compact_body, n=n),
        out_type=(jax.ShapeDtypeStruct((_NSETS * 3 * n,), jnp.float32),
                  jax.ShapeDtypeStruct((_NW * _LANES,), jnp.int32)),
        mesh=mesh,
        compiler_params=pltpu.CompilerParams(needs_layout_passes=False),
        scratch_types=[fvec, fvec, fvec, fvec, fvec, fvec,
                       pltpu.VMEM((span,), jnp.int32),
                       fvec, fvec, fvec,
                       pltpu.VMEM((_LANES,), jnp.int32),
                       pltpu.VMEM((_LANES,), jnp.int32)],
    )(p, f, m)


def _loss_body(cnt_ref, cw_ref, cd_ref, out_ref, u_ref, v_ref, colmin_ref,
               rmacc_ref, *, n, ti, tj):
    span = n // _NHALF
    b = pl.program_id(0)
    c1 = [cnt_ref[0, b * _NHALF + h] for h in range(_NHALF)]
    c2 = [cnt_ref[0, (2 + b) * _NHALF + h] for h in range(_NHALF)]

    iota = lax.broadcasted_iota(jnp.int32, (1, n), 1)
    ih = iota % span
    mrow = ih < jnp.where(iota < span, c1[0], c1[1])
    mcol = ih < jnp.where(iota < span, c2[0], c2[1])
    pw = jnp.where(mrow, cw_ref[0], 0.0)
    pdm = jnp.where(mcol, cd_ref[0], 0.0)
    sqw = jnp.sum(pw * pw, axis=0, keepdims=True)
    sqd = jnp.sum(pdm * pdm, axis=0, keepdims=True)
    ones = jnp.ones_like(sqw)
    zeros = jnp.zeros_like(sqw)
    pen_s = jnp.where(mrow, 0.0, _BIG)
    pen_d = jnp.where(mcol, 0.0, _BIG)

    def _split3(x):
        hi = x.astype(jnp.bfloat16).astype(jnp.float32)
        r1 = x - hi
        mid = r1.astype(jnp.bfloat16).astype(jnp.float32)
        lo = r1 - mid
        return hi, mid, lo

    sqw_h, sqw_m, sqw_l = _split3(sqw)
    sqd_h, sqd_m, sqd_l = _split3(sqd)

    u_ref[...] = jnp.concatenate(
        [-2.0 * pw, sqw_h, sqw_m, sqw_l, ones, ones, ones, pen_s, ones,
         zeros, zeros, zeros, zeros, zeros], axis=0).astype(jnp.bfloat16)
    v_ref[...] = jnp.concatenate(
        [pdm, ones, ones, ones, sqd_h, sqd_m, sqd_l, ones, pen_d,
         zeros, zeros, zeros, zeros, zeros], axis=0).astype(jnp.bfloat16)

    colmin_ref[...] = jnp.full((1, n), jnp.inf, dtype=jnp.float32)

    tiles_per_half_j = span // tj

    def make_iloop(half):
        base_i = half * (span // ti)

        def iloop(i, carry):
            sum_w, cnt_w = carry
            ut = u_ref[:, pl.ds((base_i + i) * ti, ti)]
            rmacc_ref[...] = jnp.full((ti, 1), jnp.inf, dtype=jnp.float32)
            for jh in range(_NHALF):
                for jj in range(tiles_per_half_j):
                    j = jh * tiles_per_half_j + jj

                    @pl.when(jj * tj < c2[jh])
                    def _(j=j):
                        vt = v_ref[:, pl.ds(j * tj, tj)]
                        dp = lax.dot_general(
                            ut, vt,
                            dimension_numbers=(((0,), (0,)), ((), ())),
                            preferred_element_type=jnp.float32,
                            precision=lax.Precision.DEFAULT)
                        cm = colmin_ref[:, pl.ds(j * tj, tj)]
                        colmin_ref[:, pl.ds(j * tj, tj)] = jnp.minimum(
                            cm, jnp.min(dp, axis=0, keepdims=True))
                        rmacc_ref[...] = jnp.minimum(
                            rmacc_ref[...],
                            jnp.min(dp, axis=1, keepdims=True))

            rmin = rmacc_ref[...]
            sel = rmin < _THR
            sum_w = sum_w + jnp.sum(jnp.where(sel, rmin, 0.0))
            cnt_w = cnt_w + jnp.sum(sel.astype(jnp.float32))
            return sum_w, cnt_w

        return iloop

    carry = (jnp.float32(0.0), jnp.float32(0.0))
    for half in range(_NHALF):
        ni = (c1[half] + ti - 1) // ti
        carry = lax.fori_loop(0, ni, make_iloop(half), carry)
    sum_w, cnt_w = carry

    colmin = colmin_ref[...]
    sel_c = colmin < _THR
    sum_c = jnp.sum(jnp.where(sel_c, colmin, 0.0))
    cnt_c = jnp.sum(sel_c.astype(jnp.float32))
    loss_b = sum_w / cnt_w + sum_c / cnt_c

    @pl.when(b == 0)
    def _():
        out_ref[0, 0] = loss_b

    @pl.when(b != 0)
    def _():
        out_ref[0, 0] = out_ref[0, 0] + loss_b


def kernel(points_src, points_dst, flows_pred, flows_gt, masks_src, masks_dst):
    del flows_gt
    bsz, n, _ = points_src.shape
    ti, tj = 512, 1024

    ps = jnp.swapaxes(points_src, 1, 2)
    pd = jnp.swapaxes(points_dst, 1, 2)
    fp = jnp.swapaxes(flows_pred, 1, 2)

    p_all = jnp.concatenate([ps, pd], axis=0)
    f_all = jnp.concatenate([fp, jnp.zeros_like(pd)], axis=0)
    m_all = jnp.concatenate([masks_src.astype(jnp.int32),
                             masks_dst.astype(jnp.int32)], axis=0)

    comp_flat, cntv = _compact(p_all.reshape(-1), f_all.reshape(-1),
                               m_all.reshape(-1), n)
    comp = comp_flat.reshape(_NSETS, 3, n)
    cnts = cntv.reshape(_NW, _LANES)[:, 0].reshape(1, _NW)

    point_spec_w = pl.BlockSpec((1, 3, n), lambda i: (i, 0, 0))
    point_spec_d = pl.BlockSpec((1, 3, n), lambda i: (2 + i, 0, 0))
    out = pl.pallas_call(
        functools.partial(_loss_body, n=n, ti=ti, tj=tj),
        grid=(bsz,),
        in_specs=[
            pl.BlockSpec((1, _NW), lambda i: (0, 0),
                         memory_space=pltpu.SMEM),
            point_spec_w,
            point_spec_d,
        ],
        out_specs=pl.BlockSpec((1, 1), lambda i: (0, 0),
                               memory_space=pltpu.SMEM),
        out_shape=jax.ShapeDtypeStruct((1, 1), jnp.float32),
        scratch_shapes=[pltpu.VMEM((16, n), jnp.bfloat16),
                        pltpu.VMEM((16, n), jnp.bfloat16),
                        pltpu.VMEM((1, n), jnp.float32),
                        pltpu.VMEM((ti, 1), jnp.float32)],
    )(cnts, comp, comp)
    return out[0, 0]


# speedup vs baseline: 5.3256x; 1.0532x over previous
import functools

import jax
import jax.numpy as jnp
from jax import lax
from jax.experimental import pallas as pl
from jax.experimental.pallas import tpu as pltpu
from jax.experimental.pallas import tpu_sc as plsc

_THR = 3.33
_BIG = 1e9
_LANES = 16
_NSETS = 4
_NHALF = 2
_NW = _NSETS * _NHALF


def _compact_body(p_hbm, f_hbm, m_hbm, outc_hbm, cnt_hbm,
                  pb0, pb1, pb2, fb0, fb1, fb2, mb, ob0, ob1, ob2, cb, sb,
                  *, n):
    span = n // _NHALF
    pbufs = (pb0, pb1, pb2)
    fbufs = (fb0, fb1, fb2)
    obufs = (ob0, ob1, ob2)
    lanes = lax.iota(jnp.int32, _LANES)

    def _cumsum16(s):
        for k in (1, 2, 4, 8):
            sb[...] = s
            g = plsc.load_gather(sb, [jnp.maximum(lanes - k, 0)])
            s = s + jnp.where(lanes >= k, g, 0)
        return s

    w = lax.axis_index("s") * 2 + lax.axis_index("c")

    @pl.when(w < _NW)
    def _():
        for k in range(3):
            pltpu.sync_copy(
                p_hbm.at[pl.ds((w // _NHALF) * 3 * n + k * n
                               + (w % _NHALF) * span, span)], pbufs[k])
            pltpu.sync_copy(
                f_hbm.at[pl.ds((w // _NHALF) * 3 * n + k * n
                               + (w % _NHALF) * span, span)], fbufs[k])
        pltpu.sync_copy(
            m_hbm.at[pl.ds((w // _NHALF) * n + (w % _NHALF) * span, span)],
            mb)

        def body(i, base):
            mv = mb[pl.ds(i * _LANES, _LANES)]
            valid = mv > 0
            pos = base + _cumsum16(mv) - 1
            for k in range(3):
                pv = (pbufs[k][pl.ds(i * _LANES, _LANES)]
                      + fbufs[k][pl.ds(i * _LANES, _LANES)])
                plsc.store_scatter(obufs[k], [pos], pv, mask=valid)
            return base + plsc.all_reduce_population_count(valid)

        cnt = lax.fori_loop(0, span // _LANES, body,
                            jnp.zeros((_LANES,), jnp.int32))
        cb[...] = cnt
        for k in range(3):
            pltpu.sync_copy(
                obufs[k],
                outc_hbm.at[pl.ds((w // _NHALF) * 3 * n + k * n
                                  + (w % _NHALF) * span, span)])
        pltpu.sync_copy(cb, cnt_hbm.at[pl.ds(w * _LANES, _LANES)])


def _compact(p, f, m, n):
    span = n // _NHALF
    mesh = plsc.VectorSubcoreMesh(core_axis_name="c", subcore_axis_name="s")
    fvec = pltpu.VMEM((span,), jnp.float32)
    return pl.kernel(
        functools.partial(_compact_body, n=n),
        out_type=(jax.ShapeDtypeStruct((_NSETS * 3 * n,), jnp.float32),
                  jax.ShapeDtypeStruct((_NW * _LANES,), jnp.int32)),
        mesh=mesh,
        compiler_params=pltpu.CompilerParams(needs_layout_passes=False),
        scratch_types=[fvec, fvec, fvec, fvec, fvec, fvec,
                       pltpu.VMEM((span,), jnp.int32),
                       fvec, fvec, fvec,
                       pltpu.VMEM((_LANES,), jnp.int32),
                       pltpu.VMEM((_LANES,), jnp.int32)],
    )(p, f, m)


def _loss_body(cnt_ref, cw_ref, cd_ref, out_ref, u_ref, v_ref, colmin_ref,
               rmacc_ref, *, n, ti, tj):
    span = n // _NHALF
    b = pl.program_id(0)
    c1 = [cnt_ref[0, b * _NHALF + h] for h in range(_NHALF)]
    c2 = [cnt_ref[0, (2 + b) * _NHALF + h] for h in range(_NHALF)]

    iota = lax.broadcasted_iota(jnp.int32, (1, n), 1)
    ih = iota % span
    mrow = ih < jnp.where(iota < span, c1[0], c1[1])
    mcol = ih < jnp.where(iota < span, c2[0], c2[1])
    pw = jnp.where(mrow, cw_ref[0], 0.0)
    pdm = jnp.where(mcol, cd_ref[0], 0.0)
    sqw = jnp.sum(pw * pw, axis=0, keepdims=True)
    sqd = jnp.sum(pdm * pdm, axis=0, keepdims=True)
    ones = jnp.ones_like(sqw)
    zeros = jnp.zeros_like(sqw)
    pen_s = jnp.where(mrow, 0.0, _BIG)
    pen_d = jnp.where(mcol, 0.0, _BIG)

    def _split3(x):
        hi = x.astype(jnp.bfloat16).astype(jnp.float32)
        r1 = x - hi
        mid = r1.astype(jnp.bfloat16).astype(jnp.float32)
        lo = r1 - mid
        return hi, mid, lo

    sqw_h, sqw_m, sqw_l = _split3(sqw)
    sqd_h, sqd_m, sqd_l = _split3(sqd)

    u_ref[...] = jnp.concatenate(
        [-2.0 * pw, sqw_h, sqw_m, sqw_l, ones, ones, ones, pen_s, ones,
         zeros, zeros, zeros, zeros, zeros], axis=0).astype(jnp.bfloat16)
    v_ref[...] = jnp.concatenate(
        [pdm, ones, ones, ones, sqd_h, sqd_m, sqd_l, ones, pen_d,
         zeros, zeros, zeros, zeros, zeros], axis=0).astype(jnp.bfloat16)

    colmin_ref[...] = jnp.full((8, n), jnp.inf, dtype=jnp.float32)

    tiles_per_half_j = span // tj

    def make_iloop(half):
        base_i = half * (span // ti)

        def iloop(i, carry):
            sum_w, cnt_w = carry
            ut = u_ref[:, pl.ds((base_i + i) * ti, ti)]
            rmacc_ref[...] = jnp.full((ti, 128), jnp.inf, dtype=jnp.float32)
            for jh in range(_NHALF):
                for jj in range(tiles_per_half_j):
                    j = jh * tiles_per_half_j + jj

                    @pl.when(jj * tj < c2[jh])
                    def _(j=j):
                        vt = v_ref[:, pl.ds(j * tj, tj)]
                        dp = lax.dot_general(
                            ut, vt,
                            dimension_numbers=(((0,), (0,)), ((), ())),
                            preferred_element_type=jnp.float32,
                            precision=lax.Precision.DEFAULT)
                        cm8 = dp[0:8, :]
                        for r in range(1, ti // 8):
                            cm8 = jnp.minimum(cm8, dp[r * 8:(r + 1) * 8, :])
                        colmin_ref[:, pl.ds(j * tj, tj)] = jnp.minimum(
                            colmin_ref[:, pl.ds(j * tj, tj)], cm8)
                        rm = dp[:, 0:128]
                        for c in range(1, tj // 128):
                            rm = jnp.minimum(rm, dp[:, c * 128:(c + 1) * 128])
                        rmacc_ref[...] = jnp.minimum(rmacc_ref[...], rm)

            rmin = jnp.min(rmacc_ref[...], axis=1, keepdims=True)
            sel = rmin < _THR
            sum_w = sum_w + jnp.sum(jnp.where(sel, rmin, 0.0))
            cnt_w = cnt_w + jnp.sum(sel.astype(jnp.float32))
            return sum_w, cnt_w

        return iloop

    carry = (jnp.float32(0.0), jnp.float32(0.0))
    for half in range(_NHALF):
        ni = (c1[half] + ti - 1) // ti
        carry = lax.fori_loop(0, ni, make_iloop(half), carry)
    sum_w, cnt_w = carry

    colmin = jnp.min(colmin_ref[...], axis=0, keepdims=True)
    sel_c = colmin < _THR
    sum_c = jnp.sum(jnp.where(sel_c, colmin, 0.0))
    cnt_c = jnp.sum(sel_c.astype(jnp.float32))
    loss_b = sum_w / cnt_w + sum_c / cnt_c

    @pl.when(b == 0)
    def _():
        out_ref[0, 0] = loss_b

    @pl.when(b != 0)
    def _():
        out_ref[0, 0] = out_ref[0, 0] + loss_b


def kernel(points_src, points_dst, flows_pred, flows_gt, masks_src, masks_dst):
    del flows_gt
    bsz, n, _ = points_src.shape
    ti, tj = 512, 1024

    ps = jnp.swapaxes(points_src, 1, 2)
    pd = jnp.swapaxes(points_dst, 1, 2)
    fp = jnp.swapaxes(flows_pred, 1, 2)

    p_all = jnp.concatenate([ps, pd], axis=0)
    f_all = jnp.concatenate([fp, jnp.zeros_like(pd)], axis=0)
    m_all = jnp.concatenate([masks_src.astype(jnp.int32),
                             masks_dst.astype(jnp.int32)], axis=0)

    comp_flat, cntv = _compact(p_all.reshape(-1), f_all.reshape(-1),
                               m_all.reshape(-1), n)
    comp = comp_flat.reshape(_NSETS, 3, n)
    cnts = cntv.reshape(_NW, _LANES)[:, 0].reshape(1, _NW)

    point_spec_w = pl.BlockSpec((1, 3, n), lambda i: (i, 0, 0))
    point_spec_d = pl.BlockSpec((1, 3, n), lambda i: (2 + i, 0, 0))
    out = pl.pallas_call(
        functools.partial(_loss_body, n=n, ti=ti, tj=tj),
        grid=(bsz,),
        in_specs=[
            pl.BlockSpec((1, _NW), lambda i: (0, 0),
                         memory_space=pltpu.SMEM),
            point_spec_w,
            point_spec_d,
        ],
        out_specs=pl.BlockSpec((1, 1), lambda i: (0, 0),
                               memory_space=pltpu.SMEM),
        out_shape=jax.ShapeDtypeStruct((1, 1), jnp.float32),
        scratch_shapes=[pltpu.VMEM((16, n), jnp.bfloat16),
                        pltpu.VMEM((16, n), jnp.bfloat16),
                        pltpu.VMEM((8, n), jnp.float32),
                        pltpu.VMEM((ti, 128), jnp.float32)],
    )(cnts, comp, comp)
    return out[0, 0]


# speedup vs baseline: 5.8988x; 1.1076x over previous
import functools

import jax
import jax.numpy as jnp
from jax import lax
from jax.experimental import pallas as pl
from jax.experimental.pallas import tpu as pltpu
from jax.experimental.pallas import tpu_sc as plsc

_THR = 3.33
_BIG = 1e9
_LANES = 16
_NSETS = 4
_NHALF = 2
_NW = _NSETS * _NHALF


def _compact_body(p_hbm, f_hbm, m_hbm, outc_hbm, cnt_hbm,
                  pb0, pb1, pb2, fb0, fb1, fb2, mb, ob0, ob1, ob2, cb, sb,
                  *, n):
    span = n // _NHALF
    pbufs = (pb0, pb1, pb2)
    fbufs = (fb0, fb1, fb2)
    obufs = (ob0, ob1, ob2)
    lanes = lax.iota(jnp.int32, _LANES)

    def _cumsum16(s):
        for k in (1, 2, 4, 8):
            sb[...] = s
            g = plsc.load_gather(sb, [jnp.maximum(lanes - k, 0)])
            s = s + jnp.where(lanes >= k, g, 0)
        return s

    w = lax.axis_index("s") * 2 + lax.axis_index("c")

    @pl.when(w < _NW)
    def _():
        for k in range(3):
            pltpu.sync_copy(
                p_hbm.at[pl.ds((w // _NHALF) * 3 * n + k * n
                               + (w % _NHALF) * span, span)], pbufs[k])
            pltpu.sync_copy(
                f_hbm.at[pl.ds((w // _NHALF) * 3 * n + k * n
                               + (w % _NHALF) * span, span)], fbufs[k])
        pltpu.sync_copy(
            m_hbm.at[pl.ds((w // _NHALF) * n + (w % _NHALF) * span, span)],
            mb)

        def body(i, base):
            mv = mb[pl.ds(i * _LANES, _LANES)]
            valid = mv > 0
            pos = base + _cumsum16(mv) - 1
            for k in range(3):
                pv = (pbufs[k][pl.ds(i * _LANES, _LANES)]
                      + fbufs[k][pl.ds(i * _LANES, _LANES)])
                plsc.store_scatter(obufs[k], [pos], pv, mask=valid)
            return base + plsc.all_reduce_population_count(valid)

        cnt = lax.fori_loop(0, span // _LANES, body,
                            jnp.zeros((_LANES,), jnp.int32))
        cb[...] = cnt
        for k in range(3):
            pltpu.sync_copy(
                obufs[k],
                outc_hbm.at[pl.ds((w // _NHALF) * 3 * n + k * n
                                  + (w % _NHALF) * span, span)])
        pltpu.sync_copy(cb, cnt_hbm.at[pl.ds(w * _LANES, _LANES)])


def _compact(p, f, m, n):
    span = n // _NHALF
    mesh = plsc.VectorSubcoreMesh(core_axis_name="c", subcore_axis_name="s")
    fvec = pltpu.VMEM((span,), jnp.float32)
    return pl.kernel(
        functools.partial(_compact_body, n=n),
        out_type=(jax.ShapeDtypeStruct((_NSETS * 3 * n,), jnp.float32),
                  jax.ShapeDtypeStruct((_NW * _LANES,), jnp.int32)),
        mesh=mesh,
        compiler_params=pltpu.CompilerParams(needs_layout_passes=False),
        scratch_types=[fvec, fvec, fvec, fvec, fvec, fvec,
                       pltpu.VMEM((span,), jnp.int32),
                       fvec, fvec, fvec,
                       pltpu.VMEM((_LANES,), jnp.int32),
                       pltpu.VMEM((_LANES,), jnp.int32)],
    )(p, f, m)


def _loss_body(cnt_ref, cw_ref, cd_ref, out_ref, u_ref, v_ref, colmin_ref,
               rmacc_ref, *, n, ti, tj):
    span = n // _NHALF
    b = pl.program_id(0)
    c1 = [cnt_ref[0, b * _NHALF + h] for h in range(_NHALF)]
    c2 = [cnt_ref[0, (2 + b) * _NHALF + h] for h in range(_NHALF)]

    iota = lax.broadcasted_iota(jnp.int32, (1, n), 1)
    ih = iota % span
    mrow = ih < jnp.where(iota < span, c1[0], c1[1])
    mcol = ih < jnp.where(iota < span, c2[0], c2[1])
    pw = jnp.where(mrow, cw_ref[0], 0.0)
    pdm = jnp.where(mcol, cd_ref[0], 0.0)
    sqw = jnp.sum(pw * pw, axis=0, keepdims=True)
    sqd = jnp.sum(pdm * pdm, axis=0, keepdims=True)
    ones = jnp.ones_like(sqw)
    zeros = jnp.zeros_like(sqw)
    pen_s = jnp.where(mrow, 0.0, _BIG)
    pen_d = jnp.where(mcol, 0.0, _BIG)

    def _split3(x):
        hi = x.astype(jnp.bfloat16).astype(jnp.float32)
        r1 = x - hi
        mid = r1.astype(jnp.bfloat16).astype(jnp.float32)
        lo = r1 - mid
        return hi, mid, lo

    sqw_h, sqw_m, sqw_l = _split3(sqw)
    sqd_h, sqd_m, sqd_l = _split3(sqd)

    u_ref[...] = jnp.concatenate(
        [-2.0 * pw, sqw_h, sqw_m, sqw_l, ones, ones, ones, pen_s, ones,
         zeros, zeros, zeros, zeros, zeros], axis=0).astype(jnp.bfloat16)
    v_ref[...] = jnp.concatenate(
        [pdm, ones, ones, ones, sqd_h, sqd_m, sqd_l, ones, pen_d,
         zeros, zeros, zeros, zeros, zeros], axis=0).astype(jnp.bfloat16)

    colmin_ref[...] = jnp.full((8, n), jnp.inf, dtype=jnp.float32)

    tiles_per_half_j = span // tj

    def make_iloop(half):
        base_i = half * (span // ti)

        def iloop(i, carry):
            sum_w, cnt_w = carry
            ut = u_ref[:, pl.ds((base_i + i) * ti, ti)]
            rmacc_ref[...] = jnp.full((ti, 128), jnp.inf, dtype=jnp.float32)
            for jh in range(_NHALF):
                for jj in range(tiles_per_half_j):
                    j = jh * tiles_per_half_j + jj

                    @pl.when(jj * tj < c2[jh])
                    def _(j=j):
                        vt = v_ref[:, pl.ds(j * tj, tj)]
                        dp = lax.dot_general(
                            ut, vt,
                            dimension_numbers=(((0,), (0,)), ((), ())),
                            preferred_element_type=jnp.float32,
                            precision=lax.Precision.DEFAULT)
                        cm8 = dp[0:8, :]
                        for r in range(1, ti // 8):
                            cm8 = jnp.minimum(cm8, dp[r * 8:(r + 1) * 8, :])
                        colmin_ref[:, pl.ds(j * tj, tj)] = jnp.minimum(
                            colmin_ref[:, pl.ds(j * tj, tj)], cm8)
                        rm = dp[:, 0:128]
                        for c in range(1, tj // 128):
                            rm = jnp.minimum(rm, dp[:, c * 128:(c + 1) * 128])
                        rmacc_ref[...] = jnp.minimum(rmacc_ref[...], rm)

            rmin = jnp.min(rmacc_ref[...], axis=1, keepdims=True)
            sel = rmin < _THR
            sum_w = sum_w + jnp.sum(jnp.where(sel, rmin, 0.0))
            cnt_w = cnt_w + jnp.sum(sel.astype(jnp.float32))
            return sum_w, cnt_w

        return iloop

    carry = (jnp.float32(0.0), jnp.float32(0.0))
    for half in range(_NHALF):
        ni = (c1[half] + ti - 1) // ti
        carry = lax.fori_loop(0, ni, make_iloop(half), carry)
    sum_w, cnt_w = carry

    colmin = jnp.min(colmin_ref[...], axis=0, keepdims=True)
    sel_c = colmin < _THR
    sum_c = jnp.sum(jnp.where(sel_c, colmin, 0.0))
    cnt_c = jnp.sum(sel_c.astype(jnp.float32))
    loss_b = sum_w / cnt_w + sum_c / cnt_c

    @pl.when(b == 0)
    def _():
        out_ref[0, 0] = loss_b

    @pl.when(b != 0)
    def _():
        out_ref[0, 0] = out_ref[0, 0] + loss_b


def kernel(points_src, points_dst, flows_pred, flows_gt, masks_src, masks_dst):
    del flows_gt
    bsz, n, _ = points_src.shape
    ti, tj = 512, 2048

    ps = jnp.swapaxes(points_src, 1, 2)
    pd = jnp.swapaxes(points_dst, 1, 2)
    fp = jnp.swapaxes(flows_pred, 1, 2)

    p_all = jnp.concatenate([ps, pd], axis=0)
    f_all = jnp.concatenate([fp, jnp.zeros_like(pd)], axis=0)
    m_all = jnp.concatenate([masks_src.astype(jnp.int32),
                             masks_dst.astype(jnp.int32)], axis=0)

    comp_flat, cntv = _compact(p_all.reshape(-1), f_all.reshape(-1),
                               m_all.reshape(-1), n)
    comp = comp_flat.reshape(_NSETS, 3, n)
    cnts = cntv.reshape(_NW, _LANES)[:, 0].reshape(1, _NW)

    point_spec_w = pl.BlockSpec((1, 3, n), lambda i: (i, 0, 0))
    point_spec_d = pl.BlockSpec((1, 3, n), lambda i: (2 + i, 0, 0))
    out = pl.pallas_call(
        functools.partial(_loss_body, n=n, ti=ti, tj=tj),
        grid=(bsz,),
        in_specs=[
            pl.BlockSpec((1, _NW), lambda i: (0, 0),
                         memory_space=pltpu.SMEM),
            point_spec_w,
            point_spec_d,
        ],
        out_specs=pl.BlockSpec((1, 1), lambda i: (0, 0),
                               memory_space=pltpu.SMEM),
        out_shape=jax.ShapeDtypeStruct((1, 1), jnp.float32),
        scratch_shapes=[pltpu.VMEM((16, n), jnp.bfloat16),
                        pltpu.VMEM((16, n), jnp.bfloat16),
                        pltpu.VMEM((8, n), jnp.float32),
                        pltpu.VMEM((ti, 128), jnp.float32)],
    )(cnts, comp, comp)
    return out[0, 0]
